# baseline (speedup 1.0000x reference)
import functools

import jax
import jax.numpy as jnp
from jax import lax
from jax.experimental import pallas as pl
from jax.experimental.pallas import tpu as pltpu
from jax.experimental.pallas import tpu_sc as plsc

EPS = 1e-8

_H = 1024
_W = 1024
_N = _H * _W
_C = 16
_L = 16
_NW = 32
_BR = 8
_BC = 256
_BPX = _BR * _BC

_SC_ROWS = 512
_TC_ROWS = _H - _SC_ROWS
_RG0 = _TC_ROWS // _BR
_NCH = _SC_ROWS // 64
_TC_BH = 64
_TC_GRID = _TC_ROWS // _TC_BH

assert _NCH % 2 == 0
assert _SC_ROWS % 64 == 0 and _TC_ROWS % _TC_BH == 0


def _rsqrt16(x, iters=2):
    i = lax.bitcast_convert_type(x, jnp.int32)
    magic = jnp.full((_L,), 0x5F3759DF, dtype=jnp.int32)
    y = lax.bitcast_convert_type(magic - (i >> 1), jnp.float32)
    for _ in range(iters):
        y = y * (1.5 - 0.5 * x * y * y)
    return y


_mesh = plsc.VectorSubcoreMesh(core_axis_name="c", subcore_axis_name="s")


@functools.partial(
    pl.kernel,
    out_type=jax.ShapeDtypeStruct((_NW, _L), jnp.float32),
    mesh=_mesh,
    scratch_types=[
        pltpu.VMEM((_C,), jnp.float32),
        pltpu.VMEM((2, _C, _BR, _BC), jnp.float32),
        pltpu.VMEM((2, _BR, _BC), jnp.float32),
        pltpu.VMEM((_L,), jnp.float32),
        pltpu.SemaphoreType.DMA,
        pltpu.SemaphoreType.DMA,
    ],
)
def _sc_loss(proj_hbm, hm_hbm, center_hbm, out_hbm, cen_v, ch_v, hm_v, acc_v,
             sem0, sem1):
    wid = lax.axis_index("s") * 2 + lax.axis_index("c")
    sems = (sem0, sem1)

    pltpu.sync_copy(center_hbm, cen_v)
    c = cen_v[...]
    cs = [c[ch] for ch in range(_C)]
    ssc = cs[0] * cs[0]
    for ch in range(1, _C):
        ssc = ssc + cs[ch] * cs[ch]
    sv = jnp.full((_L,), ssc)
    r0 = _rsqrt16(sv, iters=3)
    cen_v[...] = c / (sv * r0 + EPS)
    cn = cen_v[...]
    scn = [cn[ch] for ch in range(_C)]

    def chunk_copies(k, b):
        q = wid * _NCH + k
        r0_ = (_RG0 + (q >> 2)) * _BR
        c0_ = (q & 3) * _BC
        cps = [
            pltpu.make_async_copy(
                proj_hbm.at[ch, pl.ds(r0_, _BR), pl.ds(c0_, _BC)],
                ch_v.at[b, ch], sems[b])
            for ch in range(_C)
        ]
        cps.append(pltpu.make_async_copy(
            hm_hbm.at[pl.ds(r0_, _BR), pl.ds(c0_, _BC)], hm_v.at[b], sems[b]))
        return cps

    def compute_chunk(b, acc):
        def inner(j, acc):
            rr = j >> 4
            cc = (j & 15) * _L
            ss = None
            dot = None
            for ch in range(_C):
                v = ch_v[b, ch, rr, pl.ds(cc, _L)]
                ss = v * v if ss is None else ss + v * v
                dot = scn[ch] * v if dot is None else dot + scn[ch] * v
            d = dot * _rsqrt16(ss) - hm_v[b, rr, pl.ds(cc, _L)]
            return acc + d * d

        return lax.fori_loop(0, _BPX // _L, inner, acc, unroll=2)

    for cp in chunk_copies(0, 0):
        cp.start()

    @pl.loop(0, _NCH, step=2, init_carry=jnp.zeros((_L,), jnp.float32))
    def acc(kbase, acc):
        for b in range(2):
            k = kbase + b

            @pl.when(k + 1 < _NCH)
            def _():
                for cp in chunk_copies(k + 1, 1 - b):
                    cp.start()

            for cp in chunk_copies(k, b):
                cp.wait()
            acc = compute_chunk(b, acc)
        return acc

    acc_v[...] = acc
    pltpu.sync_copy(acc_v, out_hbm.at[wid])


def _tc_body(center_ref, f_ref, hm_ref, out_ref):
    i = pl.program_id(0)

    c = center_ref[0, :]
    cn = c / (jnp.sqrt(jnp.sum(c * c)) + EPS)

    f = f_ref[...]
    ss = jnp.sum(f * f, axis=0)
    dot = jnp.sum(f * cn[:, None, None], axis=0)
    sim = dot / (jnp.sqrt(ss) + EPS)
    d = sim - hm_ref[...]
    part = jnp.sum(d * d)

    @pl.when(i == 0)
    def _init():
        out_ref[...] = jnp.zeros_like(out_ref)

    out_ref[...] += part.reshape(1, 1)


def kernel(proj, hm, cluster_center, cluster_ind):
    center = jnp.take(cluster_center, cluster_ind, axis=0)
    center = jax.lax.stop_gradient(center)
    proj3 = proj.reshape(_C, _H, _W)
    hm2 = hm.reshape(_H, _W)

    sc_out = _sc_loss(proj3, hm2, center)

    tc_out = pl.pallas_call(
        _tc_body,
        grid=(_TC_GRID,),
        in_specs=[
            pl.BlockSpec((1, _C), lambda i: (0, 0)),
            pl.BlockSpec((_C, _TC_BH, _W), lambda i: (0, i, 0)),
            pl.BlockSpec((_TC_BH, _W), lambda i: (i, 0)),
        ],
        out_specs=pl.BlockSpec((1, 1), lambda i: (0, 0)),
        out_shape=jax.ShapeDtypeStruct((1, 1), jnp.float32),
    )(center.reshape(1, _C), proj3, hm2)

    loss = (jnp.sum(sc_out) + tc_out[0, 0]) * (1.0 / _N)
    return (loss, loss * 0.0, loss)


# speedup vs baseline: 1.0767x; 1.0767x over previous
import functools

import jax
import jax.numpy as jnp
from jax import lax
from jax.experimental import pallas as pl
from jax.experimental.pallas import tpu as pltpu
from jax.experimental.pallas import tpu_sc as plsc

EPS = 1e-8

_H = 1024
_W = 1024
_N = _H * _W
_C = 16
_L = 16
_NW = 32
_BR = 8
_BC = 256
_BPX = _BR * _BC

_SC_ROWS = 384
_TC_ROWS = _H - _SC_ROWS
_RG0 = _TC_ROWS // _BR
_NCH = _SC_ROWS // 64
_TC_BH = 64
_TC_GRID = _TC_ROWS // _TC_BH

assert _NCH % 2 == 0
assert _SC_ROWS % 64 == 0 and _TC_ROWS % _TC_BH == 0


def _rsqrt16(x, iters=2):
    i = lax.bitcast_convert_type(x, jnp.int32)
    magic = jnp.full((_L,), 0x5F3759DF, dtype=jnp.int32)
    y = lax.bitcast_convert_type(magic - (i >> 1), jnp.float32)
    for _ in range(iters):
        y = y * (1.5 - 0.5 * x * y * y)
    return y


_mesh = plsc.VectorSubcoreMesh(core_axis_name="c", subcore_axis_name="s")


@functools.partial(
    pl.kernel,
    out_type=jax.ShapeDtypeStruct((_NW, _L), jnp.float32),
    mesh=_mesh,
    scratch_types=[
        pltpu.VMEM((_C,), jnp.float32),
        pltpu.VMEM((2, _C, _BR, _BC), jnp.float32),
        pltpu.VMEM((2, _BR, _BC), jnp.float32),
        pltpu.VMEM((_L,), jnp.float32),
        pltpu.SemaphoreType.DMA,
        pltpu.SemaphoreType.DMA,
    ],
)
def _sc_loss(proj_hbm, hm_hbm, center_hbm, out_hbm, cen_v, ch_v, hm_v, acc_v,
             sem0, sem1):
    wid = lax.axis_index("s") * 2 + lax.axis_index("c")
    sems = (sem0, sem1)

    pltpu.sync_copy(center_hbm, cen_v)
    c = cen_v[...]
    cs = [c[ch] for ch in range(_C)]
    ssc = cs[0] * cs[0]
    for ch in range(1, _C):
        ssc = ssc + cs[ch] * cs[ch]
    sv = jnp.full((_L,), ssc)
    r0 = _rsqrt16(sv, iters=3)
    cen_v[...] = c / (sv * r0 + EPS)
    cn = cen_v[...]
    scn = [cn[ch] for ch in range(_C)]

    def chunk_copies(k, b):
        q = wid * _NCH + k
        r0_ = (_RG0 + (q >> 2)) * _BR
        c0_ = (q & 3) * _BC
        cps = [
            pltpu.make_async_copy(
                proj_hbm.at[ch, pl.ds(r0_, _BR), pl.ds(c0_, _BC)],
                ch_v.at[b, ch], sems[b])
            for ch in range(_C)
        ]
        cps.append(pltpu.make_async_copy(
            hm_hbm.at[pl.ds(r0_, _BR), pl.ds(c0_, _BC)], hm_v.at[b], sems[b]))
        return cps

    def compute_chunk(b, acc):
        def inner(j, acc):
            rr = j >> 4
            cc = (j & 15) * _L
            ss = None
            dot = None
            for ch in range(_C):
                v = ch_v[b, ch, rr, pl.ds(cc, _L)]
                ss = v * v if ss is None else ss + v * v
                dot = scn[ch] * v if dot is None else dot + scn[ch] * v
            d = dot * _rsqrt16(ss) - hm_v[b, rr, pl.ds(cc, _L)]
            return acc + d * d

        return lax.fori_loop(0, _BPX // _L, inner, acc, unroll=2)

    for cp in chunk_copies(0, 0):
        cp.start()

    @pl.loop(0, _NCH, step=2, init_carry=jnp.zeros((_L,), jnp.float32))
    def acc(kbase, acc):
        for b in range(2):
            k = kbase + b

            @pl.when(k + 1 < _NCH)
            def _():
                for cp in chunk_copies(k + 1, 1 - b):
                    cp.start()

            for cp in chunk_copies(k, b):
                cp.wait()
            acc = compute_chunk(b, acc)
        return acc

    acc_v[...] = acc
    pltpu.sync_copy(acc_v, out_hbm.at[wid])


def _tc_body(center_ref, f_ref, hm_ref, out_ref):
    i = pl.program_id(0)

    c = center_ref[0, :]
    cn = c / (jnp.sqrt(jnp.sum(c * c)) + EPS)

    f = f_ref[...]
    ss = jnp.sum(f * f, axis=0)
    dot = jnp.sum(f * cn[:, None, None], axis=0)
    sim = dot / (jnp.sqrt(ss) + EPS)
    d = sim - hm_ref[...]
    part = jnp.sum(d * d)

    @pl.when(i == 0)
    def _init():
        out_ref[...] = jnp.zeros_like(out_ref)

    out_ref[...] += part.reshape(1, 1)


def kernel(proj, hm, cluster_center, cluster_ind):
    center = jnp.take(cluster_center, cluster_ind, axis=0)
    center = jax.lax.stop_gradient(center)
    proj3 = proj.reshape(_C, _H, _W)
    hm2 = hm.reshape(_H, _W)

    sc_out = _sc_loss(proj3, hm2, center)

    tc_out = pl.pallas_call(
        _tc_body,
        grid=(_TC_GRID,),
        in_specs=[
            pl.BlockSpec((1, _C), lambda i: (0, 0)),
            pl.BlockSpec((_C, _TC_BH, _W), lambda i: (0, i, 0)),
            pl.BlockSpec((_TC_BH, _W), lambda i: (i, 0)),
        ],
        out_specs=pl.BlockSpec((1, 1), lambda i: (0, 0)),
        out_shape=jax.ShapeDtypeStruct((1, 1), jnp.float32),
    )(center.reshape(1, _C), proj3, hm2)

    loss = (jnp.sum(sc_out) + tc_out[0, 0]) * (1.0 / _N)
    return (loss, loss * 0.0, loss)
